# baseline (speedup 1.0000x reference)
import functools

import jax
import jax.numpy as jnp
from jax import lax
from jax.experimental import pallas as pl
from jax.experimental.pallas import tpu as pltpu
from jax.experimental.pallas import tpu_sc as plsc

_T, _C, _H = 4096, 768, 768
_E = 64
_E1 = _E + 1
_K = 2
_P = _K * _T
_P3 = _P + _T
_B = 256
_NBR = _P // _B + _E
_NPR = _NBR * _B
_NSH = _T // _B
_NB = _NBR + _NSH
_NP = _NB * _B

_NC, _NS = 2, 16
_NW = _NC * _NS

_DSUB = 64
_DCH = _P // _NW
_NDS = _DCH // _DSUB

_TPW = _T // _NW
_CSUB = 16
_NCS = _TPW // _CSUB


def _gate_body(x_ref, gw_ref, ti_ref, tw_ref):
    xb = x_ref[...]
    logits = lax.dot_general(xb, gw_ref[...], (((1,), (1,)), ((), ())),
                             preferred_element_type=jnp.float32)
    iota = lax.broadcasted_iota(jnp.int32, logits.shape, 1)
    m1 = jnp.max(logits, axis=1, keepdims=True)
    i1 = jnp.min(jnp.where(logits == m1, iota, _E), axis=1, keepdims=True)
    l2 = jnp.where(iota == i1, -jnp.inf, logits)
    m2 = jnp.max(l2, axis=1, keepdims=True)
    i2 = jnp.min(jnp.where(l2 == m2, iota, _E), axis=1, keepdims=True)
    w1 = jax.nn.sigmoid(m1 - m2)
    ti_ref[...] = jnp.concatenate([i1, i2], axis=1)
    tw_ref[...] = jnp.concatenate([w1, 1.0 - w1], axis=1)


def _gate(x, gate_w):
    rb = 1024
    return pl.pallas_call(
        _gate_body,
        grid=(_T // rb,),
        in_specs=[
            pl.BlockSpec((rb, _C), lambda i: (i, 0)),
            pl.BlockSpec((_E, _C), lambda i: (0, 0)),
        ],
        out_specs=[
            pl.BlockSpec((rb, _K), lambda i: (i, 0)),
            pl.BlockSpec((rb, _K), lambda i: (i, 0)),
        ],
        out_shape=[
            jax.ShapeDtypeStruct((_T, _K), jnp.int32),
            jax.ShapeDtypeStruct((_T, _K), jnp.float32),
        ],
    )(x, gate_w)


def _moe_body(be_ref, nb_ref, sc_ref,
              xs_ref, x_ref, wg_ref, wu_ref, wd_ref,
              sg_ref, su_ref, sd_ref, ws_ref, os_ref):
    i = pl.program_id(0)
    e = be_ref[i]
    live = i < nb_ref[0]

    def compute(xb, wg, wu, wd):
        g = jnp.dot(xb, wg, preferred_element_type=jnp.float32)
        u = jnp.dot(xb, wu, preferred_element_type=jnp.float32)
        h = g * jax.nn.sigmoid(g) * u
        return jnp.dot(h, wd, preferred_element_type=jnp.float32)

    @pl.when(jnp.logical_and(live, e < _E))
    def _():
        ob = compute(xs_ref[...], wg_ref[0], wu_ref[0], wd_ref[0])
        w = ws_ref[...]
        os_ref[...] = ob * jnp.concatenate([w] * (_C // 128), axis=1)

    @pl.when(jnp.logical_and(live, e == _E))
    def _():
        ob = compute(x_ref[...], sg_ref[...], su_ref[...], sd_ref[...])
        os_ref[...] = ob * sc_ref[0]


def _moe(be, nb, scale, xs, x, Wg, Wu, Wd, Sg, Su, Sd, ws_b):
    grid_spec = pltpu.PrefetchScalarGridSpec(
        num_scalar_prefetch=3,
        grid=(_NB,),
        in_specs=[
            pl.BlockSpec((_B, _C),
                         lambda i, be, nb, sc: (
                             jnp.minimum(i, nb[0] - _NSH - 1), 0)),
            pl.BlockSpec((_B, _C),
                         lambda i, be, nb, sc: (
                             jnp.clip(i - (nb[0] - _NSH), 0, _NSH - 1), 0)),
            pl.BlockSpec((1, _C, _H),
                         lambda i, be, nb, sc: (jnp.minimum(be[i], _E - 1), 0, 0)),
            pl.BlockSpec((1, _C, _H),
                         lambda i, be, nb, sc: (jnp.minimum(be[i], _E - 1), 0, 0)),
            pl.BlockSpec((1, _H, _C),
                         lambda i, be, nb, sc: (jnp.minimum(be[i], _E - 1), 0, 0)),
            pl.BlockSpec((_C, _H), lambda i, be, nb, sc: (0, 0)),
            pl.BlockSpec((_C, _H), lambda i, be, nb, sc: (0, 0)),
            pl.BlockSpec((_H, _C), lambda i, be, nb, sc: (0, 0)),
            pl.BlockSpec((_B, 128),
                         lambda i, be, nb, sc: (
                             jnp.minimum(i, nb[0] - _NSH - 1), 0)),
        ],
        out_specs=pl.BlockSpec(
            (_B, _C),
            lambda i, be, nb, sc: (
                jnp.where(i < nb[0], i, jnp.minimum(nb[0], _NB - 1)), 0)),
    )
    return pl.pallas_call(
        _moe_body,
        grid_spec=grid_spec,
        out_shape=jax.ShapeDtypeStruct((_NP, _C), jnp.float32),
    )(be, nb, scale, xs, x, Wg, Wu, Wd, Sg, Su, Sd, ws_b)


def _sc_dispatch(x, tok2, pp2):
    mesh = plsc.VectorSubcoreMesh(core_axis_name="c", subcore_axis_name="s")

    @functools.partial(
        pl.kernel,
        out_type=jax.ShapeDtypeStruct((_NPR, _C), jnp.float32),
        mesh=mesh,
        scratch_types=[
            pltpu.VMEM((_NDS, _DSUB), jnp.int32),
            pltpu.VMEM((_NDS, _DSUB), jnp.int32),
            pltpu.VMEM((_DSUB, _C), jnp.float32),
            pltpu.VMEM((_DSUB, _C), jnp.float32),
            pltpu.SemaphoreType.DMA,
            pltpu.SemaphoreType.DMA,
            pltpu.SemaphoreType.DMA,
        ],
    )
    def k(x_hbm, tok_hbm, pp_hbm, xs_hbm, tok_v, pp_v, r0, r1, gsem,
          ssem0, ssem1):
        wid = lax.axis_index("s") * _NC + lax.axis_index("c")
        pltpu.sync_copy(tok_hbm.at[pl.ds(wid * _NDS, _NDS)], tok_v)
        pltpu.sync_copy(pp_hbm.at[pl.ds(wid * _NDS, _NDS)], pp_v)
        rows = (r0, r1)
        ssems = (ssem0, ssem1)
        scatters = []
        for s in range(_NDS):
            buf = s % 2
            if s >= 2:
                scatters[s - 2].wait()
            pltpu.async_copy(x_hbm.at[tok_v.at[s]], rows[buf], gsem).wait()
            scatters.append(
                pltpu.async_copy(rows[buf], xs_hbm.at[pp_v.at[s]],
                                 ssems[buf]))
        scatters[_NDS - 2].wait()
        scatters[_NDS - 1].wait()

    return k(x, tok2, pp2)


def _sc_combine(os_, pos0, pos1, pos2):
    mesh = plsc.VectorSubcoreMesh(core_axis_name="c", subcore_axis_name="s")

    @functools.partial(
        pl.kernel,
        out_type=jax.ShapeDtypeStruct((_T, _C), jnp.float32),
        mesh=mesh,
        scratch_types=[
            pltpu.VMEM((_NCS, _CSUB), jnp.int32),
            pltpu.VMEM((_NCS, _CSUB), jnp.int32),
            pltpu.VMEM((_NCS, _CSUB), jnp.int32),
            pltpu.VMEM((2, _CSUB, _C), jnp.float32),
            pltpu.VMEM((2, _CSUB, _C), jnp.float32),
            pltpu.VMEM((2, _CSUB, _C), jnp.float32),
            pltpu.SemaphoreType.DMA,
            pltpu.SemaphoreType.DMA,
            pltpu.SemaphoreType.DMA,
            pltpu.SemaphoreType.DMA,
        ],
    )
    def k(os_hbm, p0_hbm, p1_hbm, p2_hbm, y_hbm, i0, i1, i2, a, b, c,
          gsem0, gsem1, ssem0, ssem1):
        wid = lax.axis_index("s") * _NC + lax.axis_index("c")
        base = wid * _TPW
        pltpu.sync_copy(p0_hbm.at[pl.ds(wid * _NCS, _NCS)], i0)
        pltpu.sync_copy(p1_hbm.at[pl.ds(wid * _NCS, _NCS)], i1)
        pltpu.sync_copy(p2_hbm.at[pl.ds(wid * _NCS, _NCS)], i2)
        gsems = (gsem0, gsem1)
        ssems = (ssem0, ssem1)

        def fire(s):
            buf = s % 2
            return (pltpu.async_copy(os_hbm.at[i0.at[s]], a.at[buf],
                                     gsems[buf]),
                    pltpu.async_copy(os_hbm.at[i1.at[s]], b.at[buf],
                                     gsems[buf]),
                    pltpu.async_copy(os_hbm.at[i2.at[s]], c.at[buf],
                                     gsems[buf]))

        gs = {0: fire(0)}
        stores = {}
        for s in range(_NCS):
            buf = s % 2
            if s + 1 < _NCS:
                if s - 1 >= 0:
                    stores[s - 1].wait()
                gs[s + 1] = fire(s + 1)
            for g in gs[s]:
                g.wait()

            def row(j, carry2, _buf=buf):
                for kk in range(_C // 16):
                    sl = pl.ds(kk * 16, 16)
                    c[_buf, j, sl] = (a[_buf, j, sl] + b[_buf, j, sl] +
                                      c[_buf, j, sl])
                return carry2

            lax.fori_loop(0, _CSUB, row, 0)
            stores[s] = pltpu.async_copy(
                c.at[buf], y_hbm.at[pl.ds(base + s * _CSUB, _CSUB)],
                ssems[buf])
        stores[_NCS - 2].wait()
        stores[_NCS - 1].wait()

    return k(os_, pos0, pos1, pos2)


def kernel(x, gate_w, Wg, Wu, Wd, Sg, Su, Sd, shared_scale):
    ti, tw = _gate(x, gate_w)

    e3 = jnp.concatenate([ti.reshape(_P),
                          jnp.full((_T,), _E, jnp.int32)])
    tw3 = jnp.concatenate([tw.reshape(_P),
                           jnp.broadcast_to(shared_scale.astype(jnp.float32),
                                            (_T,))])
    tok3 = jnp.concatenate([
        (jnp.arange(_P, dtype=jnp.int32) // _K),
        jnp.arange(_T, dtype=jnp.int32),
    ])

    oh = (e3[:, None] == jnp.arange(_E1, dtype=jnp.int32)[None, :]).astype(
        jnp.int32)
    ohr = oh.reshape(_P3 // 128, 128, _E1)
    intra = jnp.cumsum(ohr, axis=1)
    tot = intra[:, -1, :]
    coff = jnp.cumsum(tot, axis=0) - tot
    cum = (intra + coff[:, None, :]).reshape(_P3, _E1)
    counts = tot.sum(axis=0)
    pc = ((counts + _B - 1) // _B) * _B
    pend = jnp.cumsum(pc)
    poff = pend - pc
    ppos = jnp.sum(oh * (cum - 1 + poff[None, :]), axis=1).astype(jnp.int32)

    ws = jnp.zeros((_NPR,), jnp.float32).at[ppos].set(tw3, mode="drop")
    tok2 = tok3[:_P].reshape(_NW * _NDS, _DSUB)
    pp2 = ppos[:_P].reshape(_NW * _NDS, _DSUB)

    pos01 = ppos[:_P].reshape(_T, _K)
    pos0 = pos01[:, 0].reshape(_NW * _NCS, _CSUB)
    pos1 = pos01[:, 1].reshape(_NW * _NCS, _CSUB)
    pos2 = ppos[_P:].reshape(_NW * _NCS, _CSUB)

    total = pend[-1]
    nb = (total // _B).astype(jnp.int32).reshape(1)
    bs = jnp.arange(_NB, dtype=jnp.int32) * _B
    be = jnp.searchsorted(pend, bs, side="right").astype(jnp.int32)
    be = jnp.where(bs < total, be, _E)
    ws_b = jnp.broadcast_to(ws[:, None], (_NPR, 128))
    scale = shared_scale.astype(jnp.float32).reshape(1)

    xs = _sc_dispatch(x, tok2, pp2)
    os_ = _moe(be, nb, scale, xs, x, Wg, Wu, Wd, Sg, Su, Sd, ws_b)
    y = _sc_combine(os_, pos0, pos1, pos2)
    return y


# speedup vs baseline: 1.0067x; 1.0067x over previous
import functools

import jax
import jax.numpy as jnp
from jax import lax
from jax.experimental import pallas as pl
from jax.experimental.pallas import tpu as pltpu
from jax.experimental.pallas import tpu_sc as plsc

_T, _C, _H = 4096, 768, 768
_E = 64
_E1 = _E + 1
_K = 2
_P = _K * _T
_P3 = _P + _T
_B = 256
_NBR = _P // _B + _E
_NPR = _NBR * _B
_NSH = _T // _B
_NB = _NBR + _NSH
_NP = _NB * _B

_NC, _NS = 2, 16
_NW = _NC * _NS

_DSUB = 64
_DCH = _P // _NW
_NDS = _DCH // _DSUB

_TPW = _T // _NW
_CSUB = 16
_NCS = _TPW // _CSUB


def _gate_body(x_ref, gw_ref, ti_ref, tw_ref):
    xb = x_ref[...]
    logits = lax.dot_general(xb, gw_ref[...], (((1,), (1,)), ((), ())),
                             preferred_element_type=jnp.float32)
    iota = lax.broadcasted_iota(jnp.int32, logits.shape, 1)
    m1 = jnp.max(logits, axis=1, keepdims=True)
    i1 = jnp.min(jnp.where(logits == m1, iota, _E), axis=1, keepdims=True)
    l2 = jnp.where(iota == i1, -jnp.inf, logits)
    m2 = jnp.max(l2, axis=1, keepdims=True)
    i2 = jnp.min(jnp.where(l2 == m2, iota, _E), axis=1, keepdims=True)
    w1 = jax.nn.sigmoid(m1 - m2)
    ti_ref[...] = jnp.concatenate([i1, i2], axis=1)
    tw_ref[...] = jnp.concatenate([w1, 1.0 - w1], axis=1)


def _gate(x, gate_w):
    rb = 1024
    return pl.pallas_call(
        _gate_body,
        grid=(_T // rb,),
        in_specs=[
            pl.BlockSpec((rb, _C), lambda i: (i, 0)),
            pl.BlockSpec((_E, _C), lambda i: (0, 0)),
        ],
        out_specs=[
            pl.BlockSpec((rb, _K), lambda i: (i, 0)),
            pl.BlockSpec((rb, _K), lambda i: (i, 0)),
        ],
        out_shape=[
            jax.ShapeDtypeStruct((_T, _K), jnp.int32),
            jax.ShapeDtypeStruct((_T, _K), jnp.float32),
        ],
    )(x, gate_w)


def _moe_body(be_ref, nb_ref, sc_ref,
              xs_ref, x_ref, wg_ref, wu_ref, wd_ref,
              sg_ref, su_ref, sd_ref, ws_ref, os_ref):
    i = pl.program_id(0)
    e = be_ref[i]
    live = i < nb_ref[0]

    def compute(xb, wg, wu, wd):
        g = jnp.dot(xb, wg, preferred_element_type=jnp.float32)
        u = jnp.dot(xb, wu, preferred_element_type=jnp.float32)
        h = g * jax.nn.sigmoid(g) * u
        return jnp.dot(h, wd, preferred_element_type=jnp.float32)

    @pl.when(jnp.logical_and(live, e < _E))
    def _():
        ob = compute(xs_ref[...], wg_ref[0], wu_ref[0], wd_ref[0])
        w = ws_ref[...]
        os_ref[...] = ob * jnp.concatenate([w] * (_C // 128), axis=1)

    @pl.when(jnp.logical_and(live, e == _E))
    def _():
        ob = compute(x_ref[...], sg_ref[...], su_ref[...], sd_ref[...])
        os_ref[...] = ob * sc_ref[0]


def _moe(be, nb, scale, xs, x, Wg, Wu, Wd, Sg, Su, Sd, ws_b):
    grid_spec = pltpu.PrefetchScalarGridSpec(
        num_scalar_prefetch=3,
        grid=(_NB,),
        in_specs=[
            pl.BlockSpec((_B, _C),
                         lambda i, be, nb, sc: (
                             jnp.minimum(i, nb[0] - _NSH - 1), 0)),
            pl.BlockSpec((_B, _C),
                         lambda i, be, nb, sc: (
                             jnp.clip(i - (nb[0] - _NSH), 0, _NSH - 1), 0)),
            pl.BlockSpec((1, _C, _H),
                         lambda i, be, nb, sc: (jnp.minimum(be[i], _E - 1), 0, 0)),
            pl.BlockSpec((1, _C, _H),
                         lambda i, be, nb, sc: (jnp.minimum(be[i], _E - 1), 0, 0)),
            pl.BlockSpec((1, _H, _C),
                         lambda i, be, nb, sc: (jnp.minimum(be[i], _E - 1), 0, 0)),
            pl.BlockSpec((_C, _H), lambda i, be, nb, sc: (0, 0)),
            pl.BlockSpec((_C, _H), lambda i, be, nb, sc: (0, 0)),
            pl.BlockSpec((_H, _C), lambda i, be, nb, sc: (0, 0)),
            pl.BlockSpec((_B, 128),
                         lambda i, be, nb, sc: (
                             jnp.minimum(i, nb[0] - _NSH - 1), 0)),
        ],
        out_specs=pl.BlockSpec(
            (_B, _C),
            lambda i, be, nb, sc: (
                jnp.where(i < nb[0], i, jnp.minimum(nb[0], _NB - 1)), 0)),
    )
    return pl.pallas_call(
        _moe_body,
        grid_spec=grid_spec,
        out_shape=jax.ShapeDtypeStruct((_NP, _C), jnp.float32),
    )(be, nb, scale, xs, x, Wg, Wu, Wd, Sg, Su, Sd, ws_b)


def _sc_dispatch(x, tok2, pp2):
    mesh = plsc.VectorSubcoreMesh(core_axis_name="c", subcore_axis_name="s")

    @functools.partial(
        pl.kernel,
        out_type=jax.ShapeDtypeStruct((_NPR, _C), jnp.float32),
        mesh=mesh,
        scratch_types=[
            pltpu.VMEM((_NDS, _DSUB), jnp.int32),
            pltpu.VMEM((_NDS, _DSUB), jnp.int32),
            pltpu.VMEM((_DSUB, _C), jnp.float32),
            pltpu.VMEM((_DSUB, _C), jnp.float32),
            pltpu.SemaphoreType.DMA,
            pltpu.SemaphoreType.DMA,
            pltpu.SemaphoreType.DMA,
        ],
    )
    def k(x_hbm, tok_hbm, pp_hbm, xs_hbm, tok_v, pp_v, r0, r1, gsem,
          ssem0, ssem1):
        wid = lax.axis_index("s") * _NC + lax.axis_index("c")
        pltpu.sync_copy(tok_hbm.at[pl.ds(wid * _NDS, _NDS)], tok_v)
        pltpu.sync_copy(pp_hbm.at[pl.ds(wid * _NDS, _NDS)], pp_v)
        rows = (r0, r1)
        ssems = (ssem0, ssem1)
        scatters = []
        for s in range(_NDS):
            buf = s % 2
            if s >= 2:
                scatters[s - 2].wait()
            pltpu.async_copy(x_hbm.at[tok_v.at[s]], rows[buf], gsem).wait()
            scatters.append(
                pltpu.async_copy(rows[buf], xs_hbm.at[pp_v.at[s]],
                                 ssems[buf]))
        scatters[_NDS - 2].wait()
        scatters[_NDS - 1].wait()

    return k(x, tok2, pp2)


def _sc_combine(os_, pos0, pos1, pos2):
    mesh = plsc.VectorSubcoreMesh(core_axis_name="c", subcore_axis_name="s")

    @functools.partial(
        pl.kernel,
        out_type=jax.ShapeDtypeStruct((_T, _C), jnp.float32),
        mesh=mesh,
        scratch_types=[
            pltpu.VMEM((_NCS, _CSUB), jnp.int32),
            pltpu.VMEM((_NCS, _CSUB), jnp.int32),
            pltpu.VMEM((_NCS, _CSUB), jnp.int32),
            pltpu.VMEM((3, _CSUB, _C), jnp.float32),
            pltpu.VMEM((3, _CSUB, _C), jnp.float32),
            pltpu.VMEM((3, _CSUB, _C), jnp.float32),
            pltpu.SemaphoreType.DMA,
            pltpu.SemaphoreType.DMA,
            pltpu.SemaphoreType.DMA,
            pltpu.SemaphoreType.DMA,
            pltpu.SemaphoreType.DMA,
            pltpu.SemaphoreType.DMA,
        ],
    )
    def k(os_hbm, p0_hbm, p1_hbm, p2_hbm, y_hbm, i0, i1, i2, a, b, c,
          gsem0, gsem1, gsem2, ssem0, ssem1, ssem2):
        wid = lax.axis_index("s") * _NC + lax.axis_index("c")
        base = wid * _TPW
        pltpu.sync_copy(p0_hbm.at[pl.ds(wid * _NCS, _NCS)], i0)
        pltpu.sync_copy(p1_hbm.at[pl.ds(wid * _NCS, _NCS)], i1)
        pltpu.sync_copy(p2_hbm.at[pl.ds(wid * _NCS, _NCS)], i2)
        gsems = (gsem0, gsem1, gsem2)
        ssems = (ssem0, ssem1, ssem2)

        def fire(s):
            buf = s % 3
            return (pltpu.async_copy(os_hbm.at[i0.at[s]], a.at[buf],
                                     gsems[buf]),
                    pltpu.async_copy(os_hbm.at[i1.at[s]], b.at[buf],
                                     gsems[buf]),
                    pltpu.async_copy(os_hbm.at[i2.at[s]], c.at[buf],
                                     gsems[buf]))

        gs = {0: fire(0), 1: fire(1)}
        stores = {}
        for s in range(_NCS):
            buf = s % 3
            if s + 2 < _NCS:
                if s - 1 >= 0:
                    stores[s - 1].wait()
                gs[s + 2] = fire(s + 2)
            for g in gs[s]:
                g.wait()

            def row(j, carry2, _buf=buf):
                for kk in range(_C // 16):
                    sl = pl.ds(kk * 16, 16)
                    c[_buf, j, sl] = (a[_buf, j, sl] + b[_buf, j, sl] +
                                      c[_buf, j, sl])
                return carry2

            lax.fori_loop(0, _CSUB, row, 0)
            stores[s] = pltpu.async_copy(
                c.at[buf], y_hbm.at[pl.ds(base + s * _CSUB, _CSUB)],
                ssems[buf])
        stores[_NCS - 3].wait()
        stores[_NCS - 2].wait()
        stores[_NCS - 1].wait()

    return k(os_, pos0, pos1, pos2)


def kernel(x, gate_w, Wg, Wu, Wd, Sg, Su, Sd, shared_scale):
    ti, tw = _gate(x, gate_w)

    e3 = jnp.concatenate([ti.reshape(_P),
                          jnp.full((_T,), _E, jnp.int32)])
    tw3 = jnp.concatenate([tw.reshape(_P),
                           jnp.broadcast_to(shared_scale.astype(jnp.float32),
                                            (_T,))])
    tok3 = jnp.concatenate([
        (jnp.arange(_P, dtype=jnp.int32) // _K),
        jnp.arange(_T, dtype=jnp.int32),
    ])

    oh = (e3[:, None] == jnp.arange(_E1, dtype=jnp.int32)[None, :]).astype(
        jnp.int32)
    ohr = oh.reshape(_P3 // 128, 128, _E1)
    intra = jnp.cumsum(ohr, axis=1)
    tot = intra[:, -1, :]
    coff = jnp.cumsum(tot, axis=0) - tot
    cum = (intra + coff[:, None, :]).reshape(_P3, _E1)
    counts = tot.sum(axis=0)
    pc = ((counts + _B - 1) // _B) * _B
    pend = jnp.cumsum(pc)
    poff = pend - pc
    ppos = jnp.sum(oh * (cum - 1 + poff[None, :]), axis=1).astype(jnp.int32)

    ws = jnp.zeros((_NPR,), jnp.float32).at[ppos].set(tw3, mode="drop")
    tok2 = tok3[:_P].reshape(_NW * _NDS, _DSUB)
    pp2 = ppos[:_P].reshape(_NW * _NDS, _DSUB)

    pos01 = ppos[:_P].reshape(_T, _K)
    pos0 = pos01[:, 0].reshape(_NW * _NCS, _CSUB)
    pos1 = pos01[:, 1].reshape(_NW * _NCS, _CSUB)
    pos2 = ppos[_P:].reshape(_NW * _NCS, _CSUB)

    total = pend[-1]
    nb = (total // _B).astype(jnp.int32).reshape(1)
    bs = jnp.arange(_NB, dtype=jnp.int32) * _B
    be = jnp.searchsorted(pend, bs, side="right").astype(jnp.int32)
    be = jnp.where(bs < total, be, _E)
    ws_b = jnp.broadcast_to(ws[:, None], (_NPR, 128))
    scale = shared_scale.astype(jnp.float32).reshape(1)

    xs = _sc_dispatch(x, tok2, pp2)
    os_ = _moe(be, nb, scale, xs, x, Wg, Wu, Wd, Sg, Su, Sd, ws_b)
    y = _sc_combine(os_, pos0, pos1, pos2)
    return y
